# Initial kernel scaffold; baseline (speedup 1.0000x reference)
import jax
import jax.numpy as jnp
from jax.experimental import pallas as pl


def kernel(input_tensor):
    raise NotImplementedError("write your pallas kernel here")


# speedup vs baseline: 38.7221x; 38.7221x over previous
import jax
import jax.numpy as jnp
from jax import lax
from jax.experimental import pallas as pl
from jax.experimental.pallas import tpu as pltpu

_R = 16
_EPS = 1e-08


def _sliding_w(y):
    h, w = y.shape
    z = jnp.zeros((h, _R), y.dtype)
    a0 = jnp.concatenate([z, y, z], axis=1)
    a = a0
    ln = w + 2 * _R
    for s in (1, 2, 4, 8, 16):
        ln -= s
        a = a[:, :ln] + a[:, s:s + ln]
    return a[:, :w] + a0[:, 2 * _R:]


def _localnorm_kernel(x_ref, band_ref, ic_ref, o_ref):
    x = x_ref[0]
    band = band_ref[...]
    ic = ic_ref[...]
    dn = (((1,), (0,)), ((), ()))
    hx = lax.dot_general(band, x, dn, preferred_element_type=jnp.float32,
                         precision=lax.Precision.HIGHEST)
    mean = _sliding_w(hx) * ic
    c = x - mean
    hc2 = lax.dot_general(band, c * c, dn, preferred_element_type=jnp.float32,
                          precision=lax.Precision.HIGHEST)
    std = jnp.sqrt(_sliding_w(hc2) * ic)
    o_ref[0] = c / (std + _EPS)


def kernel(input_tensor):
    B, C, H, W = input_tensor.shape
    x = input_tensor.reshape(B * C, H, W)

    i = jnp.arange(H)
    band = (jnp.abs(i[:, None] - i[None, :]) <= _R).astype(jnp.float32)
    ch = (jnp.minimum(i + _R, H - 1) - jnp.maximum(i - _R, 0) + 1)
    j = jnp.arange(W)
    cw = (jnp.minimum(j + _R, W - 1) - jnp.maximum(j - _R, 0) + 1)
    inv_counts = 1.0 / (ch[:, None] * cw[None, :]).astype(jnp.float32)

    out = pl.pallas_call(
        _localnorm_kernel,
        grid=(B * C,),
        in_specs=[
            pl.BlockSpec((1, H, W), lambda b: (b, 0, 0)),
            pl.BlockSpec((H, H), lambda b: (0, 0)),
            pl.BlockSpec((H, W), lambda b: (0, 0)),
        ],
        out_specs=pl.BlockSpec((1, H, W), lambda b: (b, 0, 0)),
        out_shape=jax.ShapeDtypeStruct((B * C, H, W), jnp.float32),
        compiler_params=pltpu.CompilerParams(
            dimension_semantics=("parallel",)),
    )(x, band, inv_counts)
    return out.reshape(B, C, H, W)


# speedup vs baseline: 46.3585x; 1.1972x over previous
import jax
import jax.numpy as jnp
from jax import lax
from jax.experimental import pallas as pl
from jax.experimental.pallas import tpu as pltpu

_R = 16
_EPS = 1e-08


def _sliding_w(y):
    h, w = y.shape
    z = jnp.zeros((h, _R), y.dtype)
    a0 = jnp.concatenate([z, y, z], axis=1)
    a = a0
    ln = w + 2 * _R
    for s in (1, 2, 4, 8, 16):
        ln -= s
        a = a[:, :ln] + a[:, s:s + ln]
    return a[:, :w] + a0[:, 2 * _R:]


def _localnorm_kernel(x_ref, band_ref, ic_ref, o_ref):
    x = x_ref[0]
    band = band_ref[...]
    ic = ic_ref[...]
    dn = (((1,), (0,)), ((), ()))
    hx = lax.dot_general(band, x, dn, preferred_element_type=jnp.float32)
    mean = _sliding_w(hx) * ic
    c = x - mean
    hc2 = lax.dot_general(band, c * c, dn, preferred_element_type=jnp.float32)
    std = jnp.sqrt(_sliding_w(hc2) * ic)
    o_ref[0] = c / (std + _EPS)


def kernel(input_tensor):
    B, C, H, W = input_tensor.shape
    x = input_tensor.reshape(B * C, H, W)

    i = jnp.arange(H)
    band = (jnp.abs(i[:, None] - i[None, :]) <= _R).astype(jnp.float32)
    ch = (jnp.minimum(i + _R, H - 1) - jnp.maximum(i - _R, 0) + 1)
    j = jnp.arange(W)
    cw = (jnp.minimum(j + _R, W - 1) - jnp.maximum(j - _R, 0) + 1)
    inv_counts = 1.0 / (ch[:, None] * cw[None, :]).astype(jnp.float32)

    out = pl.pallas_call(
        _localnorm_kernel,
        grid=(B * C,),
        in_specs=[
            pl.BlockSpec((1, H, W), lambda b: (b, 0, 0)),
            pl.BlockSpec((H, H), lambda b: (0, 0)),
            pl.BlockSpec((H, W), lambda b: (0, 0)),
        ],
        out_specs=pl.BlockSpec((1, H, W), lambda b: (b, 0, 0)),
        out_shape=jax.ShapeDtypeStruct((B * C, H, W), jnp.float32),
        compiler_params=pltpu.CompilerParams(
            dimension_semantics=("parallel",)),
    )(x, band, inv_counts)
    return out.reshape(B, C, H, W)


# speedup vs baseline: 93.5698x; 2.0184x over previous
import jax
import jax.numpy as jnp
from jax import lax
from jax.experimental import pallas as pl
from jax.experimental.pallas import tpu as pltpu

_R = 16
_EPS = 1e-08
_TW = 128


def _tile_pad(y, w):
    h = y.shape[0]
    nfull, rem = divmod(w, _TW)
    z = jnp.zeros((h, _TW), y.dtype)
    pieces = [z] + [y[:, t * _TW:(t + 1) * _TW] for t in range(nfull)]
    if rem:
        tail = jnp.concatenate([y[:, nfull * _TW:],
                                jnp.zeros((h, _TW - rem), y.dtype)], axis=1)
        pieces.append(tail)
    pieces.append(z)
    return jnp.stack(pieces, axis=0)


def _untile(y3, w):
    nfull, rem = divmod(w, _TW)
    pieces = [y3[t] for t in range(nfull)]
    if rem:
        pieces.append(y3[nfull][:, :rem])
    return jnp.concatenate(pieces, axis=1)


def _box_w(y3, p0, p1, p2):
    T = y3.shape[0] - 2
    dn = (((2,), (0,)), ((), ()))
    return (lax.dot_general(y3[0:T], p0, dn, preferred_element_type=jnp.float32)
            + lax.dot_general(y3[1:T + 1], p1, dn,
                              preferred_element_type=jnp.float32)
            + lax.dot_general(y3[2:T + 2], p2, dn,
                              preferred_element_type=jnp.float32))


def _localnorm_kernel(x_ref, band_ref, p_ref, ic_ref, o_ref):
    x = x_ref[0]
    band = band_ref[...]
    p0 = p_ref[0]
    p1 = p_ref[1]
    p2 = p_ref[2]
    ic = ic_ref[...]
    h, w = x.shape
    dnh = (((0,), (0,)), ((), ()))

    wx = _untile(_box_w(_tile_pad(x.astype(jnp.bfloat16), w), p0, p1, p2), w)
    box_x = lax.dot_general(band, wx.astype(jnp.bfloat16), dnh,
                            preferred_element_type=jnp.float32)
    c = x - box_x * ic
    wc2 = _untile(_box_w(_tile_pad((c * c).astype(jnp.bfloat16), w),
                         p0, p1, p2), w)
    var = lax.dot_general(band, wc2.astype(jnp.bfloat16), dnh,
                          preferred_element_type=jnp.float32) * ic
    o_ref[0] = c / (jnp.sqrt(var) + _EPS)


def kernel(input_tensor):
    B, C, H, W = input_tensor.shape
    x = input_tensor.reshape(B * C, H, W)

    i = jnp.arange(H)
    band = (jnp.abs(i[:, None] - i[None, :]) <= _R).astype(jnp.bfloat16)
    ch = (jnp.minimum(i + _R, H - 1) - jnp.maximum(i - _R, 0) + 1)
    j = jnp.arange(W)
    cw = (jnp.minimum(j + _R, W - 1) - jnp.maximum(j - _R, 0) + 1)
    inv_counts = 1.0 / (ch[:, None] * cw[None, :]).astype(jnp.float32)

    r = jnp.arange(_TW)
    d0 = (r[:, None] - _TW) - r[None, :]
    d1 = r[:, None] - r[None, :]
    d2 = (r[:, None] + _TW) - r[None, :]
    p = jnp.stack([(jnp.abs(d) <= _R) for d in (d0, d1, d2)], axis=0)
    p = p.astype(jnp.bfloat16)

    out = pl.pallas_call(
        _localnorm_kernel,
        grid=(B * C,),
        in_specs=[
            pl.BlockSpec((1, H, W), lambda b: (b, 0, 0)),
            pl.BlockSpec((H, H), lambda b: (0, 0)),
            pl.BlockSpec((3, _TW, _TW), lambda b: (0, 0, 0)),
            pl.BlockSpec((H, W), lambda b: (0, 0)),
        ],
        out_specs=pl.BlockSpec((1, H, W), lambda b: (b, 0, 0)),
        out_shape=jax.ShapeDtypeStruct((B * C, H, W), jnp.float32),
        compiler_params=pltpu.CompilerParams(
            dimension_semantics=("parallel",)),
    )(x, band, p, inv_counts)
    return out.reshape(B, C, H, W)


# speedup vs baseline: 134.9697x; 1.4425x over previous
import jax
import jax.numpy as jnp
from jax import lax
from jax.experimental import pallas as pl
from jax.experimental.pallas import tpu as pltpu

_R = 16
_EPS = 1e-08
_TW = 256


def _tile_pad(y, w):
    h = y.shape[0]
    nfull, rem = divmod(w, _TW)
    z = jnp.zeros((h, _TW), y.dtype)
    pieces = [z] + [y[:, t * _TW:(t + 1) * _TW] for t in range(nfull)]
    if rem:
        tail = jnp.concatenate([y[:, nfull * _TW:],
                                jnp.zeros((h, _TW - rem), y.dtype)], axis=1)
        pieces.append(tail)
    pieces.append(z)
    return jnp.stack(pieces, axis=0)


def _untile(y3, w):
    nfull, rem = divmod(w, _TW)
    pieces = [y3[t] for t in range(nfull)]
    if rem:
        pieces.append(y3[nfull][:, :rem])
    return jnp.concatenate(pieces, axis=1)


def _box_w(y3, p0, p1, p2):
    T = y3.shape[0] - 2
    dn = (((2,), (0,)), ((), ()))
    return (lax.dot_general(y3[0:T], p0, dn, preferred_element_type=jnp.float32)
            + lax.dot_general(y3[1:T + 1], p1, dn,
                              preferred_element_type=jnp.float32)
            + lax.dot_general(y3[2:T + 2], p2, dn,
                              preferred_element_type=jnp.float32))


def _localnorm_kernel(x_ref, band_ref, p_ref, ic_ref, o_ref):
    band = band_ref[...]
    p0 = p_ref[0]
    p1 = p_ref[1]
    p2 = p_ref[2]
    ic = ic_ref[...]
    dnh = (((0,), (0,)), ((), ()))

    G = x_ref.shape[0]
    w = x_ref.shape[2]
    xs = [x_ref[g] for g in range(G)]
    wx = [_untile(_box_w(_tile_pad(x.astype(jnp.bfloat16), w),
                         p0, p1, p2), w) for x in xs]
    box_x = [lax.dot_general(band, y.astype(jnp.bfloat16), dnh,
                             preferred_element_type=jnp.float32) for y in wx]
    c = [x - b * ic for x, b in zip(xs, box_x)]
    wc2 = [_untile(_box_w(_tile_pad((cc * cc).astype(jnp.bfloat16), w),
                          p0, p1, p2), w) for cc in c]
    var = [lax.dot_general(band, y.astype(jnp.bfloat16), dnh,
                           preferred_element_type=jnp.float32) * ic
           for y in wc2]
    for g in range(G):
        o_ref[g] = c[g] / (jnp.sqrt(var[g]) + _EPS)


def kernel(input_tensor):
    B, C, H, W = input_tensor.shape
    x = input_tensor.reshape(B * C, H, W)

    i = jnp.arange(H)
    band = (jnp.abs(i[:, None] - i[None, :]) <= _R).astype(jnp.bfloat16)
    ch = (jnp.minimum(i + _R, H - 1) - jnp.maximum(i - _R, 0) + 1)
    j = jnp.arange(W)
    cw = (jnp.minimum(j + _R, W - 1) - jnp.maximum(j - _R, 0) + 1)
    inv_counts = 1.0 / (ch[:, None] * cw[None, :]).astype(jnp.float32)

    r = jnp.arange(_TW)
    d0 = (r[:, None] - _TW) - r[None, :]
    d1 = r[:, None] - r[None, :]
    d2 = (r[:, None] + _TW) - r[None, :]
    p = jnp.stack([(jnp.abs(d) <= _R) for d in (d0, d1, d2)], axis=0)
    p = p.astype(jnp.bfloat16)

    G = 2
    out = pl.pallas_call(
        _localnorm_kernel,
        grid=(B * C // G,),
        in_specs=[
            pl.BlockSpec((G, H, W), lambda b: (b, 0, 0)),
            pl.BlockSpec((H, H), lambda b: (0, 0)),
            pl.BlockSpec((3, _TW, _TW), lambda b: (0, 0, 0)),
            pl.BlockSpec((H, W), lambda b: (0, 0)),
        ],
        out_specs=pl.BlockSpec((G, H, W), lambda b: (b, 0, 0)),
        out_shape=jax.ShapeDtypeStruct((B * C, H, W), jnp.float32),
        compiler_params=pltpu.CompilerParams(
            dimension_semantics=("parallel",)),
    )(x, band, p, inv_counts)
    return out.reshape(B, C, H, W)


# speedup vs baseline: 161.9952x; 1.2002x over previous
import jax
import jax.numpy as jnp
from jax import lax
from jax.experimental import pallas as pl
from jax.experimental.pallas import tpu as pltpu

_R = 16
_EPS = 1e-08
_TW = 256
_G = 2


def _tiles_at(y, w, off):
    h = y.shape[0]
    pieces = []
    start = off
    while start < w:
        end = start + _TW
        if end <= w:
            pieces.append(y[:, start:end])
        else:
            pieces.append(jnp.concatenate(
                [y[:, start:], jnp.zeros((h, end - w), y.dtype)], axis=1))
        start = end
    return jnp.stack(pieces, axis=0)


def _box_w(yb, p1, mask, w):
    dn = (((2,), (0,)), ((), ()))
    ta = _tiles_at(yb, w, 0)
    oa = lax.dot_general(ta, p1, dn, preferred_element_type=jnp.float32)
    na = ta.shape[0]
    fa = jnp.concatenate(
        [oa[t] for t in range(na - 1)] + [oa[na - 1][:, :w - (na - 1) * _TW]],
        axis=1)
    tb = _tiles_at(yb, w, _TW // 2)
    nb = (w - _TW // 2) // _TW
    ob = lax.dot_general(tb[:nb], p1, dn, preferred_element_type=jnp.float32)
    h = yb.shape[0]
    fb = jnp.concatenate(
        [jnp.zeros((h, _TW // 2), jnp.float32)]
        + [ob[t] for t in range(nb)]
        + [jnp.zeros((h, w - _TW // 2 - nb * _TW), jnp.float32)], axis=1)
    return jnp.where(mask > 0, fb, fa)


def _localnorm_kernel(x_ref, band2_ref, p_ref, ic_ref, m_ref, o_ref):
    band2 = band2_ref[...]
    p1 = p_ref[...]
    ic = ic_ref[...]
    mask = m_ref[...]
    w = x_ref.shape[2]
    hh = x_ref.shape[1]
    dnh = (((0,), (0,)), ((), ()))

    xs = [x_ref[g] for g in range(_G)]
    wx = [_box_w(x.astype(jnp.bfloat16), p1, mask, w) for x in xs]
    bx = lax.dot_general(band2, jnp.concatenate(wx, 0).astype(jnp.bfloat16),
                         dnh, preferred_element_type=jnp.float32)
    c = [xs[g] - bx[g * hh:(g + 1) * hh] * ic for g in range(_G)]
    wc2 = [_box_w((cc * cc).astype(jnp.bfloat16), p1, mask, w) for cc in c]
    bv = lax.dot_general(band2, jnp.concatenate(wc2, 0).astype(jnp.bfloat16),
                         dnh, preferred_element_type=jnp.float32)
    for g in range(_G):
        var = bv[g * hh:(g + 1) * hh] * ic
        o_ref[g] = c[g] / (jnp.sqrt(var) + _EPS)


def kernel(input_tensor):
    B, C, H, W = input_tensor.shape
    x = input_tensor.reshape(B * C, H, W)

    i = jnp.arange(_G * H)
    same_img = (i[:, None] // H) == (i[None, :] // H)
    band2 = ((jnp.abs(i[:, None] - i[None, :]) <= _R) & same_img)
    band2 = band2.astype(jnp.bfloat16)
    ih = jnp.arange(H)
    ch = (jnp.minimum(ih + _R, H - 1) - jnp.maximum(ih - _R, 0) + 1)
    j = jnp.arange(W)
    cw = (jnp.minimum(j + _R, W - 1) - jnp.maximum(j - _R, 0) + 1)
    inv_counts = 1.0 / (ch[:, None] * cw[None, :]).astype(jnp.float32)

    r = jnp.arange(_TW)
    p1 = (jnp.abs(r[:, None] - r[None, :]) <= _R).astype(jnp.bfloat16)

    nb = (W - _TW // 2) // _TW
    jm = j % _TW
    bmask = (((jm >= _TW - _R) | (jm < _R)) & (j >= _TW - _R)
             & (j < nb * _TW + _TW // 2 + _R))
    bmask = jnp.broadcast_to(bmask.astype(jnp.float32), (H, W))

    out = pl.pallas_call(
        _localnorm_kernel,
        grid=(B * C // _G,),
        in_specs=[
            pl.BlockSpec((_G, H, W), lambda b: (b, 0, 0)),
            pl.BlockSpec((_G * H, _G * H), lambda b: (0, 0)),
            pl.BlockSpec((_TW, _TW), lambda b: (0, 0)),
            pl.BlockSpec((H, W), lambda b: (0, 0)),
            pl.BlockSpec((H, W), lambda b: (0, 0)),
        ],
        out_specs=pl.BlockSpec((_G, H, W), lambda b: (b, 0, 0)),
        out_shape=jax.ShapeDtypeStruct((B * C, H, W), jnp.float32),
        compiler_params=pltpu.CompilerParams(
            dimension_semantics=("parallel",)),
    )(x, band2, p1, inv_counts, bmask)
    return out.reshape(B, C, H, W)


# speedup vs baseline: 178.2736x; 1.1005x over previous
import jax
import jax.numpy as jnp
from jax import lax
from jax.experimental import pallas as pl
from jax.experimental.pallas import tpu as pltpu

_R = 16
_EPS = 1e-08
_TW = 256
_G = 2


def _tiles_at(y, w, off):
    h = y.shape[0]
    pieces = []
    start = off
    while start < w:
        end = start + _TW
        if end <= w:
            pieces.append(y[:, start:end])
        else:
            pieces.append(jnp.concatenate(
                [y[:, start:], jnp.zeros((h, end - w), y.dtype)], axis=1))
        start = end
    return jnp.stack(pieces, axis=0)


def _box_w(yb, p1, mask, w):
    dn = (((2,), (0,)), ((), ()))
    ta = _tiles_at(yb, w, 0)
    oa = lax.dot_general(ta, p1, dn, preferred_element_type=jnp.float32)
    na = ta.shape[0]
    fa = jnp.concatenate(
        [oa[t] for t in range(na - 1)] + [oa[na - 1][:, :w - (na - 1) * _TW]],
        axis=1)
    tb = _tiles_at(yb, w, _TW // 2)
    nb = (w - _TW // 2) // _TW
    ob = lax.dot_general(tb[:nb], p1, dn, preferred_element_type=jnp.float32)
    h = yb.shape[0]
    fb = jnp.concatenate(
        [jnp.zeros((h, _TW // 2), jnp.float32)]
        + [ob[t] for t in range(nb)]
        + [jnp.zeros((h, w - _TW // 2 - nb * _TW), jnp.float32)], axis=1)
    return jnp.where(mask > 0, fb, fa)


def _localnorm_kernel(x_ref, band2_ref, p_ref, ic_ref, m_ref, o_ref):
    band2 = band2_ref[...]
    p1 = p_ref[...]
    ic = ic_ref[...]
    mask = m_ref[...]
    w = x_ref.shape[2]
    hh = x_ref.shape[1]
    dnh = (((0,), (0,)), ((), ()))

    xs = [x_ref[g] for g in range(_G)]
    wx = [_box_w(x.astype(jnp.bfloat16), p1, mask, w) for x in xs]
    bx = lax.dot_general(band2, jnp.concatenate(wx, 0).astype(jnp.bfloat16),
                         dnh, preferred_element_type=jnp.float32)
    c = [xs[g] - bx[g * hh:(g + 1) * hh] * ic for g in range(_G)]
    cb = [cc.astype(jnp.bfloat16) for cc in c]
    wc2 = [_box_w(cc * cc, p1, mask, w) for cc in cb]
    bv = lax.dot_general(band2, jnp.concatenate(wc2, 0).astype(jnp.bfloat16),
                         dnh, preferred_element_type=jnp.float32)
    for g in range(_G):
        var = bv[g * hh:(g + 1) * hh] * ic
        o_ref[g] = c[g] * lax.rsqrt(var + _EPS * _EPS)


def kernel(input_tensor):
    B, C, H, W = input_tensor.shape
    x = input_tensor.reshape(B * C, H, W)

    i = jnp.arange(_G * H)
    same_img = (i[:, None] // H) == (i[None, :] // H)
    band2 = ((jnp.abs(i[:, None] - i[None, :]) <= _R) & same_img)
    band2 = band2.astype(jnp.bfloat16)
    ih = jnp.arange(H)
    ch = (jnp.minimum(ih + _R, H - 1) - jnp.maximum(ih - _R, 0) + 1)
    j = jnp.arange(W)
    cw = (jnp.minimum(j + _R, W - 1) - jnp.maximum(j - _R, 0) + 1)
    inv_counts = 1.0 / (ch[:, None] * cw[None, :]).astype(jnp.float32)

    r = jnp.arange(_TW)
    p1 = (jnp.abs(r[:, None] - r[None, :]) <= _R).astype(jnp.bfloat16)

    nb = (W - _TW // 2) // _TW
    jm = j % _TW
    bmask = (((jm >= _TW - _R) | (jm < _R)) & (j >= _TW - _R)
             & (j < nb * _TW + _TW // 2 + _R))
    bmask = jnp.broadcast_to(bmask.astype(jnp.float32), (H, W))

    out = pl.pallas_call(
        _localnorm_kernel,
        grid=(B * C // _G,),
        in_specs=[
            pl.BlockSpec((_G, H, W), lambda b: (b, 0, 0)),
            pl.BlockSpec((_G * H, _G * H), lambda b: (0, 0)),
            pl.BlockSpec((_TW, _TW), lambda b: (0, 0)),
            pl.BlockSpec((H, W), lambda b: (0, 0)),
            pl.BlockSpec((H, W), lambda b: (0, 0)),
        ],
        out_specs=pl.BlockSpec((_G, H, W), lambda b: (b, 0, 0)),
        out_shape=jax.ShapeDtypeStruct((B * C, H, W), jnp.float32),
        compiler_params=pltpu.CompilerParams(
            dimension_semantics=("parallel",)),
    )(x, band2, p1, inv_counts, bmask)
    return out.reshape(B, C, H, W)


# speedup vs baseline: 196.6651x; 1.1032x over previous
import jax
import jax.numpy as jnp
from jax import lax
from jax.experimental import pallas as pl
from jax.experimental.pallas import tpu as pltpu

_R = 16
_EPS = 1e-08
_TW = 256
_G = 2
_F8 = jnp.float8_e4m3fn


def _tiles_at(y, w, off):
    h = y.shape[0]
    pieces = []
    start = off
    while start < w:
        end = start + _TW
        if end <= w:
            pieces.append(y[:, start:end])
        else:
            pieces.append(jnp.concatenate(
                [y[:, start:], jnp.zeros((h, end - w), y.dtype)], axis=1))
        start = end
    return jnp.stack(pieces, axis=0)


def _box_w(yb, p1, mask, w):
    dn = (((2,), (0,)), ((), ()))
    ta = _tiles_at(yb, w, 0)
    oa = lax.dot_general(ta, p1, dn, preferred_element_type=jnp.float32)
    na = ta.shape[0]
    fa = jnp.concatenate(
        [oa[t] for t in range(na - 1)] + [oa[na - 1][:, :w - (na - 1) * _TW]],
        axis=1)
    tb = _tiles_at(yb, w, _TW // 2)
    nb = (w - _TW // 2) // _TW
    ob = lax.dot_general(tb[:nb], p1, dn, preferred_element_type=jnp.float32)
    h = yb.shape[0]
    fb = jnp.concatenate(
        [jnp.zeros((h, _TW // 2), jnp.float32)]
        + [ob[t] for t in range(nb)]
        + [jnp.zeros((h, w - _TW // 2 - nb * _TW), jnp.float32)], axis=1)
    return jnp.where(mask > 0, fb, fa)


def _localnorm_kernel(x_ref, band2_ref, p_ref, ic_ref, m_ref, o_ref):
    band2 = band2_ref[...]
    p1 = p_ref[...]
    ic = ic_ref[...]
    mask = m_ref[...]
    w = x_ref.shape[2]
    hh = x_ref.shape[1]
    dnh = (((0,), (0,)), ((), ()))

    xs = [x_ref[g] for g in range(_G)]
    wx = [_box_w(x.astype(_F8), p1, mask, w) for x in xs]
    bx = lax.dot_general(band2, jnp.concatenate(wx, 0).astype(_F8),
                         dnh, preferred_element_type=jnp.float32)
    c = [xs[g] - bx[g * hh:(g + 1) * hh] * ic for g in range(_G)]
    cb = [cc.astype(jnp.bfloat16) for cc in c]
    wc2 = [_box_w((cc * cc).astype(_F8), p1, mask, w) for cc in cb]
    bv = lax.dot_general(band2, jnp.concatenate(wc2, 0).astype(_F8),
                         dnh, preferred_element_type=jnp.float32)
    for g in range(_G):
        var = bv[g * hh:(g + 1) * hh] * ic
        o_ref[g] = c[g] * lax.rsqrt(var + _EPS * _EPS)


def kernel(input_tensor):
    B, C, H, W = input_tensor.shape
    x = input_tensor.reshape(B * C, H, W)

    i = jnp.arange(_G * H)
    same_img = (i[:, None] // H) == (i[None, :] // H)
    band2 = ((jnp.abs(i[:, None] - i[None, :]) <= _R) & same_img)
    band2 = band2.astype(_F8)
    ih = jnp.arange(H)
    ch = (jnp.minimum(ih + _R, H - 1) - jnp.maximum(ih - _R, 0) + 1)
    j = jnp.arange(W)
    cw = (jnp.minimum(j + _R, W - 1) - jnp.maximum(j - _R, 0) + 1)
    inv_counts = 1.0 / (ch[:, None] * cw[None, :]).astype(jnp.float32)

    r = jnp.arange(_TW)
    p1 = (jnp.abs(r[:, None] - r[None, :]) <= _R).astype(_F8)

    nb = (W - _TW // 2) // _TW
    jm = j % _TW
    bmask = (((jm >= _TW - _R) | (jm < _R)) & (j >= _TW - _R)
             & (j < nb * _TW + _TW // 2 + _R))
    bmask = jnp.broadcast_to(bmask.astype(jnp.float32), (H, W))

    out = pl.pallas_call(
        _localnorm_kernel,
        grid=(B * C // _G,),
        in_specs=[
            pl.BlockSpec((_G, H, W), lambda b: (b, 0, 0)),
            pl.BlockSpec((_G * H, _G * H), lambda b: (0, 0)),
            pl.BlockSpec((_TW, _TW), lambda b: (0, 0)),
            pl.BlockSpec((H, W), lambda b: (0, 0)),
            pl.BlockSpec((H, W), lambda b: (0, 0)),
        ],
        out_specs=pl.BlockSpec((_G, H, W), lambda b: (b, 0, 0)),
        out_shape=jax.ShapeDtypeStruct((B * C, H, W), jnp.float32),
        compiler_params=pltpu.CompilerParams(
            dimension_semantics=("parallel",)),
    )(x, band2, p1, inv_counts, bmask)
    return out.reshape(B, C, H, W)
